# baseline (speedup 1.0000x reference)
import functools

import jax
import jax.numpy as jnp
from jax import lax
from jax.experimental import pallas as pl
from jax.experimental.pallas import tpu as pltpu
from jax.experimental.pallas import tpu_sc as plsc

_B, _S, _D = 1, 2048, 1024
_E, _TOPK, _DFF = 8, 2, 2048
_T = _B * _S
_BLK = 128
_NB = (_T * _TOPK) // _BLK + _E
_NP = _NB * _BLK
_NEG = -1e30


def _router_body(x_ref, gw_ref, out_ref, cnt_ref, carry_ref):
    pid = pl.program_id(0)

    @pl.when(pid == 0)
    def _init():
        carry_ref[...] = jnp.zeros_like(carry_ref)

    xb = x_ref[...]
    logits = lax.dot_general(xb, gw_ref[...], (((1,), (1,)), ((), ())),
                             preferred_element_type=jnp.float32)
    col = lax.broadcasted_iota(jnp.int32, logits.shape, 1)
    valid = col < _E
    ml = jnp.where(valid, logits, _NEG)
    m1 = jnp.max(ml, axis=1, keepdims=True)
    e1 = jnp.min(jnp.where((ml == m1) & valid, col, 128), axis=1, keepdims=True)
    ml2 = jnp.where(col == e1, _NEG, ml)
    m2 = jnp.max(ml2, axis=1, keepdims=True)
    e2 = jnp.min(jnp.where((ml2 == m2) & valid, col, 128), axis=1, keepdims=True)
    w1n = jax.nn.sigmoid(m1 - m2)
    w2n = jax.nn.sigmoid(m2 - m1)

    oh1 = jnp.where(col == e1, 1.0, 0.0)
    oh2 = jnp.where(col == e2, 1.0, 0.0)
    oh = oh1 + oh2
    r_i = lax.broadcasted_iota(jnp.int32, (oh.shape[0], oh.shape[0]), 0)
    c_i = lax.broadcasted_iota(jnp.int32, (oh.shape[0], oh.shape[0]), 1)
    tri = jnp.where(r_i > c_i, 1.0, 0.0)
    excl = lax.dot_general(tri, oh, (((1,), (0,)), ((), ())),
                           preferred_element_type=jnp.float32,
                           precision=lax.Precision.HIGHEST)
    tot = excl + carry_ref[0:1, :]
    rank1 = jnp.sum(tot * oh1, axis=1, keepdims=True)
    rank2 = jnp.sum(tot * oh2, axis=1, keepdims=True)
    new_counts = carry_ref[0:1, :] + jnp.sum(oh, axis=0, keepdims=True)
    carry_ref[0:1, :] = new_counts

    out = logits
    out = jnp.where(col == _E + 0, w1n, out)
    out = jnp.where(col == _E + 1, w2n, out)
    out = jnp.where(col == _E + 2, e1.astype(jnp.float32), out)
    out = jnp.where(col == _E + 3, e2.astype(jnp.float32), out)
    out = jnp.where(col == _E + 4, rank1, out)
    out = jnp.where(col == _E + 5, rank2, out)
    out_ref[...] = out
    cnt_ref[...] = jnp.broadcast_to(new_counts, cnt_ref.shape)


def _run_router(x2d, gate_w):
    gwp = jnp.zeros((128, _D), jnp.float32).at[:_E].set(gate_w)
    rb = 256
    return pl.pallas_call(
        _router_body,
        grid=(_T // rb,),
        in_specs=[
            pl.BlockSpec((rb, _D), lambda i: (i, 0)),
            pl.BlockSpec((128, _D), lambda i: (0, 0)),
        ],
        out_specs=[
            pl.BlockSpec((rb, 128), lambda i: (i, 0)),
            pl.BlockSpec((8, 128), lambda i: (0, 0)),
        ],
        out_shape=[
            jax.ShapeDtypeStruct((_T, 128), jnp.float32),
            jax.ShapeDtypeStruct((8, 128), jnp.float32),
        ],
        scratch_shapes=[pltpu.VMEM((8, 128), jnp.float32)],
    )(x2d, gwp)


def _make_sc_scatter_x(d):
    info = plsc.get_sparse_core_info()
    nc, ns = info.num_cores, info.num_subcores
    nw = nc * ns
    tpw = _T // nw
    mesh = plsc.VectorSubcoreMesh(core_axis_name="c", subcore_axis_name="s")

    def body(x_hbm, p1_hbm, p2_hbm, out_hbm, i1_v, i2_v, rows_v, sem):
        wid = lax.axis_index("s") * nc + lax.axis_index("c")
        base = wid * tpw
        pltpu.sync_copy(x_hbm.at[pl.ds(base, tpw)], rows_v)
        pltpu.sync_copy(p1_hbm.at[pl.ds(base, tpw)], i1_v)
        pltpu.sync_copy(p2_hbm.at[pl.ds(base, tpw)], i2_v)
        c1 = pltpu.async_copy(rows_v, out_hbm.at[i1_v], sem)
        c2 = pltpu.async_copy(rows_v, out_hbm.at[i2_v], sem)
        c1.wait()
        c2.wait()

    return pl.kernel(
        body,
        out_type=jax.ShapeDtypeStruct((_NP, d), jnp.float32),
        mesh=mesh,
        scratch_types=[
            pltpu.VMEM((tpw,), jnp.int32),
            pltpu.VMEM((tpw,), jnp.int32),
            pltpu.VMEM((tpw, d), jnp.float32),
            pltpu.SemaphoreType.DMA,
        ],
    )


def _w_copies(w1_hbm, w2_hbm, w3_hbm, w1s, w2s, w3s, sems, e, slot):
    return (
        pltpu.make_async_copy(w1_hbm.at[e], w1s.at[slot], sems.at[slot, 0]),
        pltpu.make_async_copy(w2_hbm.at[e], w2s.at[slot], sems.at[slot, 1]),
        pltpu.make_async_copy(w3_hbm.at[e], w3s.at[slot], sems.at[slot, 2]),
    )


def _moe_body(be_ref, xs_ref, w1_hbm, w2_hbm, w3_hbm, out_ref,
              w1s, w2s, w3s, sems):
    i = pl.program_id(0)
    e = be_ref[i]
    prev = be_ref[jnp.maximum(i - 1, 0)]
    is_first = jnp.logical_or(i == 0, e != prev)
    slot = e % 2

    @pl.when(i == 0)
    def _prime():
        for c in _w_copies(w1_hbm, w2_hbm, w3_hbm, w1s, w2s, w3s, sems, 0, 0):
            c.start()
        for c in _w_copies(w1_hbm, w2_hbm, w3_hbm, w1s, w2s, w3s, sems, 1, 1):
            c.start()

    @pl.when(is_first)
    def _arrive():
        for c in _w_copies(w1_hbm, w2_hbm, w3_hbm, w1s, w2s, w3s, sems,
                           e, slot):
            c.wait()

    @pl.when(jnp.logical_and(is_first,
                             jnp.logical_and(i > 0, e < _E - 1)))
    def _prefetch_next():
        for c in _w_copies(w1_hbm, w2_hbm, w3_hbm, w1s, w2s, w3s, sems,
                           e + 1, 1 - slot):
            c.start()

    x = xs_ref[...]
    h = lax.dot_general(x, w1s[slot], (((1,), (1,)), ((), ())),
                        preferred_element_type=jnp.float32)
    g = lax.dot_general(x, w3s[slot], (((1,), (1,)), ((), ())),
                        preferred_element_type=jnp.float32)
    act = h * jax.nn.sigmoid(h) * g
    out_ref[...] = lax.dot_general(act, w2s[slot], (((1,), (1,)), ((), ())),
                                   preferred_element_type=jnp.float32)


def _run_moe(xs, W1, W2, W3, block_expert):
    grid_spec = pltpu.PrefetchScalarGridSpec(
        num_scalar_prefetch=1,
        grid=(_NB,),
        in_specs=[
            pl.BlockSpec((_BLK, _D), lambda i, be: (i, 0)),
            pl.BlockSpec(memory_space=pl.ANY),
            pl.BlockSpec(memory_space=pl.ANY),
            pl.BlockSpec(memory_space=pl.ANY),
        ],
        out_specs=pl.BlockSpec((_BLK, _D), lambda i, be: (i, 0)),
        scratch_shapes=[
            pltpu.VMEM((2, _DFF, _D), jnp.float32),
            pltpu.VMEM((2, _D, _DFF), jnp.float32),
            pltpu.VMEM((2, _DFF, _D), jnp.float32),
            pltpu.SemaphoreType.DMA((2, 3)),
        ],
    )
    return pl.pallas_call(
        _moe_body,
        grid_spec=grid_spec,
        out_shape=jax.ShapeDtypeStruct((_NP, _D), jnp.float32),
        compiler_params=pltpu.CompilerParams(
            vmem_limit_bytes=128 * 1024 * 1024,
        ),
    )(block_expert, xs, W1, W2, W3)


def _make_sc_combine(d):
    info = plsc.get_sparse_core_info()
    nc, ns = info.num_cores, info.num_subcores
    nw = nc * ns
    tpw = _T // nw
    ch = 32
    nch = tpw // ch
    nsl = d // 16
    mesh = plsc.VectorSubcoreMesh(core_axis_name="c", subcore_axis_name="s")

    def body(tab_hbm, p1_hbm, p2_hbm, w1_hbm, w2_hbm, out_hbm,
             i1_v, i2_v, w1_v, w2_v, r1_v, r2_v, sem):
        wid = lax.axis_index("s") * nc + lax.axis_index("c")
        base = wid * tpw
        for ci in range(nch):
            cb = base + ci * ch
            pltpu.sync_copy(p1_hbm.at[pl.ds(cb, ch)], i1_v)
            pltpu.sync_copy(p2_hbm.at[pl.ds(cb, ch)], i2_v)
            pltpu.sync_copy(w1_hbm.at[pl.ds(cb, ch)], w1_v)
            pltpu.sync_copy(w2_hbm.at[pl.ds(cb, ch)], w2_v)
            c1 = pltpu.async_copy(tab_hbm.at[i1_v], r1_v, sem)
            c2 = pltpu.async_copy(tab_hbm.at[i2_v], r2_v, sem)
            c1.wait()
            c2.wait()

            def tok(t, carry):
                w1s = w1_v[t]
                w2s = w2_v[t]
                for s in range(nsl):
                    sl = pl.ds(s * 16, 16)
                    r1_v[t, sl] = r1_v[t, sl] * w1s + r2_v[t, sl] * w2s
                return carry

            lax.fori_loop(0, ch, tok, 0)
            pltpu.sync_copy(r1_v, out_hbm.at[pl.ds(cb, ch)])

    return pl.kernel(
        body,
        out_type=jax.ShapeDtypeStruct((_T, d), jnp.float32),
        mesh=mesh,
        scratch_types=[
            pltpu.VMEM((ch,), jnp.int32),
            pltpu.VMEM((ch,), jnp.int32),
            pltpu.VMEM((ch, 16), jnp.float32),
            pltpu.VMEM((ch, 16), jnp.float32),
            pltpu.VMEM((ch, d), jnp.float32),
            pltpu.VMEM((ch, d), jnp.float32),
            pltpu.SemaphoreType.DMA,
        ],
    )


def kernel(hidden_states, gate_w, W1, W2, W3):
    x2d = hidden_states.reshape(_T, _D).astype(jnp.float32)

    routed, cnts = _run_router(x2d, gate_w)
    router_logits = routed[:, :_E]
    w1n = routed[:, _E:_E + 1]
    w2n = routed[:, _E + 1:_E + 2]
    e1 = routed[:, _E + 2].astype(jnp.int32)
    e2 = routed[:, _E + 3].astype(jnp.int32)
    rank1 = routed[:, _E + 4].astype(jnp.int32)
    rank2 = routed[:, _E + 5].astype(jnp.int32)

    counts = cnts[0, :_E].astype(jnp.int32)
    padded = jnp.maximum(((counts + _BLK - 1) // _BLK) * _BLK, _BLK)
    ps = jnp.concatenate([jnp.zeros((1,), jnp.int32),
                          jnp.cumsum(padded)[:-1].astype(jnp.int32)])
    bounds = (ps // _BLK).astype(jnp.int32)
    bidx = jnp.arange(_NB, dtype=jnp.int32)
    block_expert = jnp.sum(
        (bidx[:, None] >= bounds[None, 1:]).astype(jnp.int32), axis=1)
    p1 = (ps[e1] + rank1).astype(jnp.int32)
    p2 = (ps[e2] + rank2).astype(jnp.int32)

    xs = _make_sc_scatter_x(_D)(x2d, p1, p2)
    out_sorted = _run_moe(xs, W1, W2, W3, block_expert)

    w1b = jnp.broadcast_to(w1n, (_T, 16))
    w2b = jnp.broadcast_to(w2n, (_T, 16))
    final2d = _make_sc_combine(_D)(out_sorted, p1, p2, w1b, w2b)

    return (final2d.reshape(_B, _S, _D), router_logits)


# speedup vs baseline: 1.4180x; 1.4180x over previous
import functools

import jax
import jax.numpy as jnp
from jax import lax
from jax.experimental import pallas as pl
from jax.experimental.pallas import tpu as pltpu
from jax.experimental.pallas import tpu_sc as plsc

_B, _S, _D = 1, 2048, 1024
_E, _TOPK, _DFF = 8, 2, 2048
_T = _B * _S
_BLK = 256
_NB = (_T * _TOPK) // _BLK + _E
_NP = _NB * _BLK
_NEG = -1e30


def _router_body(x_ref, gw_ref, out_ref, cnt_ref, w1b_ref, w2b_ref,
                 carry_ref):
    pid = pl.program_id(0)

    @pl.when(pid == 0)
    def _init():
        carry_ref[...] = jnp.zeros_like(carry_ref)

    xb = x_ref[...]
    logits = lax.dot_general(xb, gw_ref[...], (((1,), (1,)), ((), ())),
                             preferred_element_type=jnp.float32)
    col = lax.broadcasted_iota(jnp.int32, logits.shape, 1)
    valid = col < _E
    ml = jnp.where(valid, logits, _NEG)
    m1 = jnp.max(ml, axis=1, keepdims=True)
    e1 = jnp.min(jnp.where((ml == m1) & valid, col, 128), axis=1, keepdims=True)
    ml2 = jnp.where(col == e1, _NEG, ml)
    m2 = jnp.max(ml2, axis=1, keepdims=True)
    e2 = jnp.min(jnp.where((ml2 == m2) & valid, col, 128), axis=1, keepdims=True)
    w1n = jax.nn.sigmoid(m1 - m2)
    w2n = jax.nn.sigmoid(m2 - m1)

    oh1 = jnp.where(col == e1, 1.0, 0.0)
    oh2 = jnp.where(col == e2, 1.0, 0.0)
    oh = oh1 + oh2
    r_i = lax.broadcasted_iota(jnp.int32, (oh.shape[0], oh.shape[0]), 0)
    c_i = lax.broadcasted_iota(jnp.int32, (oh.shape[0], oh.shape[0]), 1)
    tri = jnp.where(r_i > c_i, 1.0, 0.0)
    excl = lax.dot_general(tri, oh, (((1,), (0,)), ((), ())),
                           preferred_element_type=jnp.float32,
                           precision=lax.Precision.HIGHEST)
    tot = excl + carry_ref[0:1, :]
    rank1 = jnp.sum(tot * oh1, axis=1, keepdims=True)
    rank2 = jnp.sum(tot * oh2, axis=1, keepdims=True)
    new_counts = carry_ref[0:1, :] + jnp.sum(oh, axis=0, keepdims=True)
    carry_ref[0:1, :] = new_counts

    out = logits
    out = jnp.where(col == _E + 0, w1n, out)
    out = jnp.where(col == _E + 1, w2n, out)
    out = jnp.where(col == _E + 2, e1.astype(jnp.float32), out)
    out = jnp.where(col == _E + 3, e2.astype(jnp.float32), out)
    out = jnp.where(col == _E + 4, rank1, out)
    out = jnp.where(col == _E + 5, rank2, out)
    out_ref[...] = out
    cnt_ref[...] = jnp.broadcast_to(new_counts, cnt_ref.shape)
    w1b_ref[...] = jnp.broadcast_to(w1n, w1b_ref.shape)
    w2b_ref[...] = jnp.broadcast_to(w2n, w2b_ref.shape)


def _run_router(x2d, gate_w):
    gwp = jnp.zeros((128, _D), jnp.float32).at[:_E].set(gate_w)
    rb = 256
    return pl.pallas_call(
        _router_body,
        grid=(_T // rb,),
        in_specs=[
            pl.BlockSpec((rb, _D), lambda i: (i, 0)),
            pl.BlockSpec((128, _D), lambda i: (0, 0)),
        ],
        out_specs=[
            pl.BlockSpec((rb, 128), lambda i: (i, 0)),
            pl.BlockSpec((8, 128), lambda i: (0, 0)),
            pl.BlockSpec((rb, 16), lambda i: (i, 0)),
            pl.BlockSpec((rb, 16), lambda i: (i, 0)),
        ],
        out_shape=[
            jax.ShapeDtypeStruct((_T, 128), jnp.float32),
            jax.ShapeDtypeStruct((8, 128), jnp.float32),
            jax.ShapeDtypeStruct((_T, 16), jnp.float32),
            jax.ShapeDtypeStruct((_T, 16), jnp.float32),
        ],
        scratch_shapes=[pltpu.VMEM((8, 128), jnp.float32)],
    )(x2d, gwp)


def _make_sc_scatter_x(d):
    info = plsc.get_sparse_core_info()
    nc, ns = info.num_cores, info.num_subcores
    nw = nc * ns
    tpw = _T // nw
    mesh = plsc.VectorSubcoreMesh(core_axis_name="c", subcore_axis_name="s")

    def body(x_hbm, p1_hbm, p2_hbm, out_hbm, i1_v, i2_v, rows_v, sem):
        wid = lax.axis_index("s") * nc + lax.axis_index("c")
        base = wid * tpw
        pltpu.sync_copy(x_hbm.at[pl.ds(base, tpw)], rows_v)
        pltpu.sync_copy(p1_hbm.at[pl.ds(base, tpw)], i1_v)
        pltpu.sync_copy(p2_hbm.at[pl.ds(base, tpw)], i2_v)
        c1 = pltpu.async_copy(rows_v, out_hbm.at[i1_v], sem)
        c2 = pltpu.async_copy(rows_v, out_hbm.at[i2_v], sem)
        c1.wait()
        c2.wait()

    return pl.kernel(
        body,
        out_type=jax.ShapeDtypeStruct((_NP, d), jnp.float32),
        mesh=mesh,
        scratch_types=[
            pltpu.VMEM((tpw,), jnp.int32),
            pltpu.VMEM((tpw,), jnp.int32),
            pltpu.VMEM((tpw, d), jnp.float32),
            pltpu.SemaphoreType.DMA,
        ],
    )


def _w_copies(w1_hbm, w2_hbm, w3_hbm, w1s, w2s, w3s, sems, e, slot):
    return (
        pltpu.make_async_copy(w1_hbm.at[e], w1s.at[slot], sems.at[slot, 0]),
        pltpu.make_async_copy(w2_hbm.at[e], w2s.at[slot], sems.at[slot, 1]),
        pltpu.make_async_copy(w3_hbm.at[e], w3s.at[slot], sems.at[slot, 2]),
    )


def _moe_body(be_ref, xs_ref, w1_hbm, w2_hbm, w3_hbm, out_ref,
              w1s, w2s, w3s, sems):
    i = pl.program_id(0)
    e = be_ref[i]
    prev = be_ref[jnp.maximum(i - 1, 0)]
    is_first = jnp.logical_or(i == 0, e != prev)
    slot = e % 2

    @pl.when(i == 0)
    def _prime():
        for c in _w_copies(w1_hbm, w2_hbm, w3_hbm, w1s, w2s, w3s, sems, 0, 0):
            c.start()
        for c in _w_copies(w1_hbm, w2_hbm, w3_hbm, w1s, w2s, w3s, sems, 1, 1):
            c.start()

    @pl.when(is_first)
    def _arrive():
        for c in _w_copies(w1_hbm, w2_hbm, w3_hbm, w1s, w2s, w3s, sems,
                           e, slot):
            c.wait()

    @pl.when(jnp.logical_and(is_first,
                             jnp.logical_and(i > 0, e < _E - 1)))
    def _prefetch_next():
        for c in _w_copies(w1_hbm, w2_hbm, w3_hbm, w1s, w2s, w3s, sems,
                           e + 1, 1 - slot):
            c.start()

    x = xs_ref[...]
    h = lax.dot_general(x, w1s[slot], (((1,), (1,)), ((), ())),
                        preferred_element_type=jnp.float32)
    g = lax.dot_general(x, w3s[slot], (((1,), (1,)), ((), ())),
                        preferred_element_type=jnp.float32)
    act = h * jax.nn.sigmoid(h) * g
    out_ref[...] = lax.dot_general(act, w2s[slot], (((1,), (1,)), ((), ())),
                                   preferred_element_type=jnp.float32)


def _run_moe(xs, W1, W2, W3, block_expert):
    grid_spec = pltpu.PrefetchScalarGridSpec(
        num_scalar_prefetch=1,
        grid=(_NB,),
        in_specs=[
            pl.BlockSpec((_BLK, _D), lambda i, be: (i, 0)),
            pl.BlockSpec(memory_space=pl.ANY),
            pl.BlockSpec(memory_space=pl.ANY),
            pl.BlockSpec(memory_space=pl.ANY),
        ],
        out_specs=pl.BlockSpec((_BLK, _D), lambda i, be: (i, 0)),
        scratch_shapes=[
            pltpu.VMEM((2, _DFF, _D), jnp.float32),
            pltpu.VMEM((2, _D, _DFF), jnp.float32),
            pltpu.VMEM((2, _DFF, _D), jnp.float32),
            pltpu.SemaphoreType.DMA((2, 3)),
        ],
    )
    return pl.pallas_call(
        _moe_body,
        grid_spec=grid_spec,
        out_shape=jax.ShapeDtypeStruct((_NP, _D), jnp.float32),
        compiler_params=pltpu.CompilerParams(
            vmem_limit_bytes=128 * 1024 * 1024,
        ),
    )(block_expert, xs, W1, W2, W3)


def _make_sc_combine(d):
    info = plsc.get_sparse_core_info()
    nc, ns = info.num_cores, info.num_subcores
    nw = nc * ns
    tpw = _T // nw
    ch = 32
    nch = tpw // ch
    nsl = d // 16
    mesh = plsc.VectorSubcoreMesh(core_axis_name="c", subcore_axis_name="s")

    def body(tab_hbm, p1_hbm, p2_hbm, w1_hbm, w2_hbm, out_hbm,
             i1_v, i2_v, w1_v, w2_v, r1_v, r2_v, sem):
        wid = lax.axis_index("s") * nc + lax.axis_index("c")
        base = wid * tpw
        for ci in range(nch):
            cb = base + ci * ch
            pltpu.sync_copy(p1_hbm.at[pl.ds(cb, ch)], i1_v)
            pltpu.sync_copy(p2_hbm.at[pl.ds(cb, ch)], i2_v)
            pltpu.sync_copy(w1_hbm.at[pl.ds(cb, ch)], w1_v)
            pltpu.sync_copy(w2_hbm.at[pl.ds(cb, ch)], w2_v)
            c1 = pltpu.async_copy(tab_hbm.at[i1_v], r1_v, sem)
            c2 = pltpu.async_copy(tab_hbm.at[i2_v], r2_v, sem)
            c1.wait()
            c2.wait()

            def tok(t, carry):
                w1s = w1_v[t]
                w2s = w2_v[t]
                for s in range(nsl):
                    sl = pl.ds(s * 16, 16)
                    r1_v[t, sl] = r1_v[t, sl] * w1s + r2_v[t, sl] * w2s
                return carry

            lax.fori_loop(0, ch, tok, 0)
            pltpu.sync_copy(r1_v, out_hbm.at[pl.ds(cb, ch)])

    return pl.kernel(
        body,
        out_type=jax.ShapeDtypeStruct((_T, d), jnp.float32),
        mesh=mesh,
        scratch_types=[
            pltpu.VMEM((ch,), jnp.int32),
            pltpu.VMEM((ch,), jnp.int32),
            pltpu.VMEM((ch, 16), jnp.float32),
            pltpu.VMEM((ch, 16), jnp.float32),
            pltpu.VMEM((ch, d), jnp.float32),
            pltpu.VMEM((ch, d), jnp.float32),
            pltpu.SemaphoreType.DMA,
        ],
    )


def kernel(hidden_states, gate_w, W1, W2, W3):
    x2d = hidden_states.reshape(_T, _D).astype(jnp.float32)

    routed, cnts, w1b, w2b = _run_router(x2d, gate_w)
    router_logits = routed[:, :_E]
    e1 = routed[:, _E + 2].astype(jnp.int32)
    e2 = routed[:, _E + 3].astype(jnp.int32)
    rank1 = routed[:, _E + 4].astype(jnp.int32)
    rank2 = routed[:, _E + 5].astype(jnp.int32)

    counts = cnts[0, :_E].astype(jnp.int32)
    padded = jnp.maximum(((counts + _BLK - 1) // _BLK) * _BLK, _BLK)
    ps = jnp.concatenate([jnp.zeros((1,), jnp.int32),
                          jnp.cumsum(padded)[:-1].astype(jnp.int32)])
    bounds = (ps // _BLK).astype(jnp.int32)
    bidx = jnp.arange(_NB, dtype=jnp.int32)
    block_expert = jnp.sum(
        (bidx[:, None] >= bounds[None, 1:]).astype(jnp.int32), axis=1)
    p1 = (ps[e1] + rank1).astype(jnp.int32)
    p2 = (ps[e2] + rank2).astype(jnp.int32)

    xs = _make_sc_scatter_x(_D)(x2d, p1, p2)
    out_sorted = _run_moe(xs, W1, W2, W3, block_expert)

    final2d = _make_sc_combine(_D)(out_sorted, p1, p2, w1b, w2b)

    return (final2d.reshape(_B, _S, _D), router_logits)


# speedup vs baseline: 1.4453x; 1.0192x over previous
import functools

import jax
import jax.numpy as jnp
from jax import lax
from jax.experimental import pallas as pl
from jax.experimental.pallas import tpu as pltpu
from jax.experimental.pallas import tpu_sc as plsc

_B, _S, _D = 1, 2048, 1024
_E, _TOPK, _DFF = 8, 2, 2048
_T = _B * _S
_BLK = 256
_NB = (_T * _TOPK) // _BLK + _E
_NP = _NB * _BLK
_NEG = -1e30


def _router_body(x_ref, gw_ref, out_ref, cnt_ref, w1b_ref, w2b_ref,
                 carry_ref):
    pid = pl.program_id(0)

    @pl.when(pid == 0)
    def _init():
        carry_ref[...] = jnp.zeros_like(carry_ref)

    xb = x_ref[...]
    logits = lax.dot_general(xb, gw_ref[...], (((1,), (1,)), ((), ())),
                             preferred_element_type=jnp.float32)
    col = lax.broadcasted_iota(jnp.int32, logits.shape, 1)
    valid = col < _E
    ml = jnp.where(valid, logits, _NEG)
    m1 = jnp.max(ml, axis=1, keepdims=True)
    e1 = jnp.min(jnp.where((ml == m1) & valid, col, 128), axis=1, keepdims=True)
    ml2 = jnp.where(col == e1, _NEG, ml)
    m2 = jnp.max(ml2, axis=1, keepdims=True)
    e2 = jnp.min(jnp.where((ml2 == m2) & valid, col, 128), axis=1, keepdims=True)
    w1n = jax.nn.sigmoid(m1 - m2)
    w2n = jax.nn.sigmoid(m2 - m1)

    oh1 = jnp.where(col == e1, 1.0, 0.0)
    oh2 = jnp.where(col == e2, 1.0, 0.0)
    oh = oh1 + oh2
    r_i = lax.broadcasted_iota(jnp.int32, (oh.shape[0], oh.shape[0]), 0)
    c_i = lax.broadcasted_iota(jnp.int32, (oh.shape[0], oh.shape[0]), 1)
    tri = jnp.where(r_i > c_i, 1.0, 0.0)
    excl = lax.dot_general(tri, oh, (((1,), (0,)), ((), ())),
                           preferred_element_type=jnp.float32,
                           precision=lax.Precision.HIGHEST)
    tot = excl + carry_ref[0:1, :]
    rank1 = jnp.sum(tot * oh1, axis=1, keepdims=True)
    rank2 = jnp.sum(tot * oh2, axis=1, keepdims=True)
    new_counts = carry_ref[0:1, :] + jnp.sum(oh, axis=0, keepdims=True)
    carry_ref[0:1, :] = new_counts

    out = logits
    out = jnp.where(col == _E + 0, w1n, out)
    out = jnp.where(col == _E + 1, w2n, out)
    out = jnp.where(col == _E + 2, e1.astype(jnp.float32), out)
    out = jnp.where(col == _E + 3, e2.astype(jnp.float32), out)
    out = jnp.where(col == _E + 4, rank1, out)
    out = jnp.where(col == _E + 5, rank2, out)
    out_ref[...] = out
    cnt_ref[...] = jnp.broadcast_to(new_counts, cnt_ref.shape)
    w1b_ref[...] = jnp.broadcast_to(w1n, w1b_ref.shape)
    w2b_ref[...] = jnp.broadcast_to(w2n, w2b_ref.shape)


def _run_router(x2d, gate_w):
    gwp = jnp.zeros((128, _D), jnp.float32).at[:_E].set(gate_w)
    rb = 256
    return pl.pallas_call(
        _router_body,
        grid=(_T // rb,),
        in_specs=[
            pl.BlockSpec((rb, _D), lambda i: (i, 0)),
            pl.BlockSpec((128, _D), lambda i: (0, 0)),
        ],
        out_specs=[
            pl.BlockSpec((rb, 128), lambda i: (i, 0)),
            pl.BlockSpec((8, 128), lambda i: (0, 0)),
            pl.BlockSpec((rb, 16), lambda i: (i, 0)),
            pl.BlockSpec((rb, 16), lambda i: (i, 0)),
        ],
        out_shape=[
            jax.ShapeDtypeStruct((_T, 128), jnp.float32),
            jax.ShapeDtypeStruct((8, 128), jnp.float32),
            jax.ShapeDtypeStruct((_T, 16), jnp.float32),
            jax.ShapeDtypeStruct((_T, 16), jnp.float32),
        ],
        scratch_shapes=[pltpu.VMEM((8, 128), jnp.float32)],
    )(x2d, gwp)


def _make_sc_scatter_x(d):
    info = plsc.get_sparse_core_info()
    nc, ns = info.num_cores, info.num_subcores
    nw = nc * ns
    tpw = _T // nw
    mesh = plsc.VectorSubcoreMesh(core_axis_name="c", subcore_axis_name="s")

    def body(x_hbm, p1_hbm, p2_hbm, out_hbm, i1_v, i2_v, rows_v, sem):
        wid = lax.axis_index("s") * nc + lax.axis_index("c")
        base = wid * tpw
        pltpu.sync_copy(x_hbm.at[pl.ds(base, tpw)], rows_v)
        pltpu.sync_copy(p1_hbm.at[pl.ds(base, tpw)], i1_v)
        pltpu.sync_copy(p2_hbm.at[pl.ds(base, tpw)], i2_v)
        c1 = pltpu.async_copy(rows_v, out_hbm.at[i1_v], sem)
        c2 = pltpu.async_copy(rows_v, out_hbm.at[i2_v], sem)
        c1.wait()
        c2.wait()

    return pl.kernel(
        body,
        out_type=jax.ShapeDtypeStruct((_NP, d), jnp.float32),
        mesh=mesh,
        scratch_types=[
            pltpu.VMEM((tpw,), jnp.int32),
            pltpu.VMEM((tpw,), jnp.int32),
            pltpu.VMEM((tpw, d), jnp.float32),
            pltpu.SemaphoreType.DMA,
        ],
    )


def _w_copies(w1_hbm, w2_hbm, w3_hbm, w1s, w2s, w3s, sems, e, slot):
    return (
        pltpu.make_async_copy(w1_hbm.at[e], w1s.at[slot], sems.at[slot, 0]),
        pltpu.make_async_copy(w2_hbm.at[e], w2s.at[slot], sems.at[slot, 1]),
        pltpu.make_async_copy(w3_hbm.at[e], w3s.at[slot], sems.at[slot, 2]),
    )


def _moe_body(be_ref, xs_ref, w1_hbm, w2_hbm, w3_hbm, out_ref,
              w1s, w2s, w3s, sems):
    i = pl.program_id(0)
    e = be_ref[i]
    prev = be_ref[jnp.maximum(i - 1, 0)]
    is_first = jnp.logical_or(i == 0, e != prev)
    slot = e % 2

    @pl.when(i == 0)
    def _prime():
        for c in _w_copies(w1_hbm, w2_hbm, w3_hbm, w1s, w2s, w3s, sems, 0, 0):
            c.start()
        for c in _w_copies(w1_hbm, w2_hbm, w3_hbm, w1s, w2s, w3s, sems, 1, 1):
            c.start()

    @pl.when(is_first)
    def _arrive():
        for c in _w_copies(w1_hbm, w2_hbm, w3_hbm, w1s, w2s, w3s, sems,
                           e, slot):
            c.wait()

    @pl.when(jnp.logical_and(is_first,
                             jnp.logical_and(i > 0, e < _E - 1)))
    def _prefetch_next():
        for c in _w_copies(w1_hbm, w2_hbm, w3_hbm, w1s, w2s, w3s, sems,
                           e + 1, 1 - slot):
            c.start()

    x = xs_ref[...]
    h = lax.dot_general(x, w1s[slot], (((1,), (1,)), ((), ())),
                        preferred_element_type=jnp.float32)
    g = lax.dot_general(x, w3s[slot], (((1,), (1,)), ((), ())),
                        preferred_element_type=jnp.float32)
    act = h * jax.nn.sigmoid(h) * g
    out_ref[...] = lax.dot_general(act, w2s[slot], (((1,), (1,)), ((), ())),
                                   preferred_element_type=jnp.float32)


def _run_moe(xs, W1, W2, W3, block_expert):
    grid_spec = pltpu.PrefetchScalarGridSpec(
        num_scalar_prefetch=1,
        grid=(_NB,),
        in_specs=[
            pl.BlockSpec((_BLK, _D), lambda i, be: (i, 0)),
            pl.BlockSpec(memory_space=pl.ANY),
            pl.BlockSpec(memory_space=pl.ANY),
            pl.BlockSpec(memory_space=pl.ANY),
        ],
        out_specs=pl.BlockSpec((_BLK, _D), lambda i, be: (i, 0)),
        scratch_shapes=[
            pltpu.VMEM((2, _DFF, _D), jnp.float32),
            pltpu.VMEM((2, _D, _DFF), jnp.float32),
            pltpu.VMEM((2, _DFF, _D), jnp.float32),
            pltpu.SemaphoreType.DMA((2, 3)),
        ],
    )
    return pl.pallas_call(
        _moe_body,
        grid_spec=grid_spec,
        out_shape=jax.ShapeDtypeStruct((_NP, _D), jnp.float32),
        compiler_params=pltpu.CompilerParams(
            vmem_limit_bytes=128 * 1024 * 1024,
        ),
    )(block_expert, xs, W1, W2, W3)


def _make_sc_combine(d):
    info = plsc.get_sparse_core_info()
    nc, ns = info.num_cores, info.num_subcores
    nw = nc * ns
    tpw = _T // nw
    ch = 16
    nch = tpw // ch
    nsl = d // 16
    mesh = plsc.VectorSubcoreMesh(core_axis_name="c", subcore_axis_name="s")

    def body(tab_hbm, p1_hbm, p2_hbm, w1_hbm, w2_hbm, out_hbm,
             i1a, i1b, i2a, i2b, w1a, w1b, w2a, w2b,
             r1a, r1b, r2a, r2b, sem_a, sem_b):
        wid = lax.axis_index("s") * nc + lax.axis_index("c")
        base = wid * tpw
        i1 = (i1a, i1b)
        i2 = (i2a, i2b)
        w1 = (w1a, w1b)
        w2 = (w2a, w2b)
        r1 = (r1a, r1b)
        r2 = (r2a, r2b)
        sems = (sem_a, sem_b)

        def stage(ci):
            p = ci % 2
            cb = base + ci * ch
            pltpu.sync_copy(p1_hbm.at[pl.ds(cb, ch)], i1[p])
            pltpu.sync_copy(p2_hbm.at[pl.ds(cb, ch)], i2[p])
            pltpu.sync_copy(w1_hbm.at[pl.ds(cb, ch)], w1[p])
            pltpu.sync_copy(w2_hbm.at[pl.ds(cb, ch)], w2[p])
            g1 = pltpu.async_copy(tab_hbm.at[i1[p]], r1[p], sems[p])
            g2 = pltpu.async_copy(tab_hbm.at[i2[p]], r2[p], sems[p])
            return g1, g2

        inflight = stage(0)
        for ci in range(nch):
            p = ci % 2
            cb = base + ci * ch
            g1, g2 = inflight
            if ci + 1 < nch:
                inflight = stage(ci + 1)
            g1.wait()
            g2.wait()

            def tok(t, carry):
                w1s = w1[p][t]
                w2s = w2[p][t]
                for s in range(nsl):
                    sl = pl.ds(s * 16, 16)
                    r1[p][t, sl] = r1[p][t, sl] * w1s + r2[p][t, sl] * w2s
                return carry

            lax.fori_loop(0, ch, tok, 0)
            pltpu.sync_copy(r1[p], out_hbm.at[pl.ds(cb, ch)])

    return pl.kernel(
        body,
        out_type=jax.ShapeDtypeStruct((_T, d), jnp.float32),
        mesh=mesh,
        scratch_types=(
            [pltpu.VMEM((ch,), jnp.int32)] * 4
            + [pltpu.VMEM((ch, 16), jnp.float32)] * 4
            + [pltpu.VMEM((ch, d), jnp.float32)] * 4
            + [pltpu.SemaphoreType.DMA] * 2
        ),
    )


def kernel(hidden_states, gate_w, W1, W2, W3):
    x2d = hidden_states.reshape(_T, _D).astype(jnp.float32)

    routed, cnts, w1b, w2b = _run_router(x2d, gate_w)
    router_logits = routed[:, :_E]
    e1 = routed[:, _E + 2].astype(jnp.int32)
    e2 = routed[:, _E + 3].astype(jnp.int32)
    rank1 = routed[:, _E + 4].astype(jnp.int32)
    rank2 = routed[:, _E + 5].astype(jnp.int32)

    counts = cnts[0, :_E].astype(jnp.int32)
    padded = jnp.maximum(((counts + _BLK - 1) // _BLK) * _BLK, _BLK)
    ps = jnp.concatenate([jnp.zeros((1,), jnp.int32),
                          jnp.cumsum(padded)[:-1].astype(jnp.int32)])
    bounds = (ps // _BLK).astype(jnp.int32)
    bidx = jnp.arange(_NB, dtype=jnp.int32)
    block_expert = jnp.sum(
        (bidx[:, None] >= bounds[None, 1:]).astype(jnp.int32), axis=1)
    p1 = (ps[e1] + rank1).astype(jnp.int32)
    p2 = (ps[e2] + rank2).astype(jnp.int32)

    xs = _make_sc_scatter_x(_D)(x2d, p1, p2)
    out_sorted = _run_moe(xs, W1, W2, W3, block_expert)

    final2d = _make_sc_combine(_D)(out_sorted, p1, p2, w1b, w2b)

    return (final2d.reshape(_B, _S, _D), router_logits)
